# baseline (speedup 1.0000x reference)
import jax
import jax.numpy as jnp
from jax import lax
from jax.experimental import pallas as pl
from jax.experimental.pallas import tpu as pltpu
from jax.experimental.pallas import tpu_sc as plsc

_N_PAIRS = 16384
_N_ROWS = 100000
_D = 64
_NC = 2
_NS = 16
_NW = _NC * _NS
_PER_W = _N_PAIRS // _NW
_CHUNK = 128
_NCHUNK = _PER_W // _CHUNK
_L = 16
_NSIDE_CHUNKS = 2 * _NCHUNK


def _sc_scores_body(u_pos_t, i_pos_t, u_neg_t, i_neg_t,
                    uh_pos, ih_pos, up_pos, ip_pos,
                    uh_neg, ih_neg, up_neg, ip_neg,
                    pos_out, neg_out,
                    uh_v, ih_v, upar_v, ipar_v,
                    urows_v, irows_v, scores_v, sem0, sem1):
    wid = lax.axis_index("s") * _NC + lax.axis_index("c")
    base = wid * _PER_W
    pltpu.sync_copy(uh_pos.at[wid], uh_v.at[0])
    pltpu.sync_copy(ih_pos.at[wid], ih_v.at[0])
    pltpu.sync_copy(up_pos.at[wid], upar_v.at[0])
    pltpu.sync_copy(ip_pos.at[wid], ipar_v.at[0])
    pltpu.sync_copy(uh_neg.at[wid], uh_v.at[1])
    pltpu.sync_copy(ih_neg.at[wid], ih_v.at[1])
    pltpu.sync_copy(up_neg.at[wid], upar_v.at[1])
    pltpu.sync_copy(ip_neg.at[wid], ipar_v.at[1])

    sems = (sem0, sem1)
    sides = ((u_pos_t, i_pos_t, pos_out), (u_neg_t, i_neg_t, neg_out))
    lane_iota = lax.iota(jnp.int32, _L)
    perms = {s: lane_iota ^ s for s in (8, 4, 2, 1)}

    pending = {}

    def issue(c):
        side, j = c // _NCHUNK, c % _NCHUNK
        slot = c % 2
        u_t, i_t, _ = sides[side]
        d1 = pltpu.async_copy(u_t.at[uh_v.at[side, j]],
                              urows_v.at[slot], sems[slot])
        d2 = pltpu.async_copy(i_t.at[ih_v.at[side, j]],
                              irows_v.at[slot], sems[slot])
        pending[c] = (d1, d2)

    issue(0)
    issue(1)
    for c in range(_NSIDE_CHUNKS):
        side, j = c // _NCHUNK, c % _NCHUNK
        slot = c % 2
        out = sides[side][2]
        for d in pending.pop(c):
            d.wait()
        urows = urows_v.at[slot]
        irows = irows_v.at[slot]

        def group_body(g, carry, side=side, j=j, urows=urows, irows=irows):
            paru = upar_v[side, j, pl.ds(g * _L, _L)]
            pari = ipar_v[side, j, pl.ds(g * _L, _L)]
            for l in range(_L):
                p = g * _L + l
                rows16 = jnp.full((_L,), p, jnp.int32)
                sel = jnp.full((_L,), l, jnp.int32)
                pu = paru[sel]
                pi = pari[sel]
                acc = None
                for kc in range(_D // _L):
                    ccol = kc * _L + lane_iota
                    uvk = plsc.load_gather(urows, [rows16, pu + ccol])
                    ivk = plsc.load_gather(irows, [rows16, pi + ccol])
                    prod = uvk * ivk
                    acc = prod if acc is None else acc + prod
                for s in (8, 4, 2, 1):
                    acc = acc + acc[perms[s]]
                plsc.store_scatter(scores_v, [rows16], acc,
                                   mask=lane_iota == l)
            return carry

        lax.fori_loop(0, _CHUNK // _L, group_body, 0)
        pltpu.sync_copy(scores_v, out.at[pl.ds(base + j * _CHUNK, _CHUNK)])
        if c + 2 < _NSIDE_CHUNKS:
            issue(c + 2)


def _sc_scores(u_pos_t, i_pos_t, u_neg_t, i_neg_t, idx_arrays):
    mesh = plsc.VectorSubcoreMesh(core_axis_name="c", subcore_axis_name="s",
                                  num_cores=_NC, num_subcores=_NS)
    fn = pl.kernel(
        _sc_scores_body,
        out_type=[jax.ShapeDtypeStruct((_N_PAIRS,), jnp.float32),
                  jax.ShapeDtypeStruct((_N_PAIRS,), jnp.float32)],
        mesh=mesh,
        compiler_params=pltpu.CompilerParams(needs_layout_passes=False,
                                             use_tc_tiling_on_sc=True),
        scratch_types=[
            pltpu.VMEM((2, _NCHUNK, _CHUNK), jnp.int32),
            pltpu.VMEM((2, _NCHUNK, _CHUNK), jnp.int32),
            pltpu.VMEM((2, _NCHUNK, _CHUNK), jnp.int32),
            pltpu.VMEM((2, _NCHUNK, _CHUNK), jnp.int32),
            pltpu.VMEM((2, _CHUNK, 2 * _D), jnp.float32),
            pltpu.VMEM((2, _CHUNK, 2 * _D), jnp.float32),
            pltpu.VMEM((_CHUNK,), jnp.float32),
            pltpu.SemaphoreType.DMA,
            pltpu.SemaphoreType.DMA,
        ],
    )
    return fn(u_pos_t, i_pos_t, u_neg_t, i_neg_t, *idx_arrays)


def _loss_body(pos_ref, neg_ref, out_ref):
    pos = pos_ref[...]
    neg = neg_ref[...]
    total = jnp.sum(jax.nn.log_sigmoid(pos) + jax.nn.log_sigmoid(-neg))
    out_ref[0, 0] = -total / _N_PAIRS


def _loss(pos_scores, neg_scores):
    p = pos_scores.reshape(_N_PAIRS // 128, 128)
    n = neg_scores.reshape(_N_PAIRS // 128, 128)
    out = pl.pallas_call(
        _loss_body,
        out_shape=jax.ShapeDtypeStruct((1, 1), jnp.float32),
        out_specs=pl.BlockSpec(memory_space=pltpu.SMEM),
    )(p, n)
    return out[0, 0]


def _split_idx(pairs, col):
    ids = pairs[:, col]
    half = (ids // 2).astype(jnp.int32).reshape(_NW, _NCHUNK, _CHUNK)
    par = ((ids & 1) * _D).astype(jnp.int32).reshape(_NW, _NCHUNK, _CHUNK)
    return half, par


def kernel(user_emb_pos, item_emb_pos, user_emb_neg, item_emb_neg,
           positive_pairs, negative_pairs):
    uh_pos, up_pos = _split_idx(positive_pairs, 0)
    ih_pos, ip_pos = _split_idx(positive_pairs, 1)
    uh_neg, up_neg = _split_idx(negative_pairs, 0)
    ih_neg, ip_neg = _split_idx(negative_pairs, 1)
    wide = lambda t: t.reshape(_N_ROWS // 2, 2 * _D)
    pos_s, neg_s = _sc_scores(
        wide(user_emb_pos), wide(item_emb_pos),
        wide(user_emb_neg), wide(item_emb_neg),
        (uh_pos, ih_pos, up_pos, ip_pos,
         uh_neg, ih_neg, up_neg, ip_neg))
    return _loss(pos_s, neg_s)


# speedup vs baseline: 1.0349x; 1.0349x over previous
import jax
import jax.numpy as jnp
from jax import lax
from jax.experimental import pallas as pl
from jax.experimental.pallas import tpu as pltpu
from jax.experimental.pallas import tpu_sc as plsc

_N_PAIRS = 16384
_D = 64
_NC = 2
_NS = 16
_NW = _NC * _NS
_PER_W = _N_PAIRS // _NW
_CHUNK = 128
_NCHUNK = _PER_W // _CHUNK
_L = 16
_NSIDE_CHUNKS = 2 * _NCHUNK


def _sc_scores_body(u_pos_t, i_pos_t, u_neg_t, i_neg_t,
                    uidx_pos, iidx_pos, uidx_neg, iidx_neg,
                    pos_out, neg_out,
                    uidx_v, iidx_v,
                    urows_v, irows_v, scores_v, sem0, sem1):
    wid = lax.axis_index("s") * _NC + lax.axis_index("c")
    base = wid * _PER_W
    pltpu.sync_copy(uidx_pos.at[wid], uidx_v.at[0])
    pltpu.sync_copy(iidx_pos.at[wid], iidx_v.at[0])
    pltpu.sync_copy(uidx_neg.at[wid], uidx_v.at[1])
    pltpu.sync_copy(iidx_neg.at[wid], iidx_v.at[1])

    sems = (sem0, sem1)
    sides = ((u_pos_t, i_pos_t, pos_out), (u_neg_t, i_neg_t, neg_out))
    lane_iota = lax.iota(jnp.int32, _L)
    perms = {s: lane_iota ^ s for s in (8, 4, 2, 1)}

    pending = {}

    def issue(c):
        side, j = c // _NCHUNK, c % _NCHUNK
        slot = c % 2
        u_t, i_t, _ = sides[side]
        d1 = pltpu.async_copy(u_t.at[uidx_v.at[side, j]],
                              urows_v.at[slot], sems[slot])
        d2 = pltpu.async_copy(i_t.at[iidx_v.at[side, j]],
                              irows_v.at[slot], sems[slot])
        pending[c] = (d1, d2)

    issue(0)
    issue(1)
    for c in range(_NSIDE_CHUNKS):
        side, j = c // _NCHUNK, c % _NCHUNK
        slot = c % 2
        out = sides[side][2]
        for d in pending.pop(c):
            d.wait()
        urows = urows_v.at[slot]
        irows = irows_v.at[slot]

        def group_body(g, carry, urows=urows, irows=irows):
            for l in range(_L):
                p = g * _L + l
                rows16 = jnp.full((_L,), p, jnp.int32)
                acc = None
                for kc in range(_D // _L):
                    ccol = kc * _L + lane_iota
                    uvk = plsc.load_gather(urows, [rows16, ccol])
                    ivk = plsc.load_gather(irows, [rows16, ccol])
                    prod = uvk * ivk
                    acc = prod if acc is None else acc + prod
                for s in (8, 4, 2, 1):
                    acc = acc + acc[perms[s]]
                plsc.store_scatter(scores_v, [rows16], acc,
                                   mask=lane_iota == l)
            return carry

        lax.fori_loop(0, _CHUNK // _L, group_body, 0)
        pltpu.sync_copy(scores_v, out.at[pl.ds(base + j * _CHUNK, _CHUNK)])
        if c + 2 < _NSIDE_CHUNKS:
            issue(c + 2)


def _sc_scores(u_pos_t, i_pos_t, u_neg_t, i_neg_t, idx_arrays):
    mesh = plsc.VectorSubcoreMesh(core_axis_name="c", subcore_axis_name="s",
                                  num_cores=_NC, num_subcores=_NS)
    fn = pl.kernel(
        _sc_scores_body,
        out_type=[jax.ShapeDtypeStruct((_N_PAIRS,), jnp.float32),
                  jax.ShapeDtypeStruct((_N_PAIRS,), jnp.float32)],
        mesh=mesh,
        compiler_params=pltpu.CompilerParams(needs_layout_passes=False,
                                             use_tc_tiling_on_sc=False),
        scratch_types=[
            pltpu.VMEM((2, _NCHUNK, _CHUNK), jnp.int32),
            pltpu.VMEM((2, _NCHUNK, _CHUNK), jnp.int32),
            pltpu.VMEM((2, _CHUNK, _D), jnp.float32),
            pltpu.VMEM((2, _CHUNK, _D), jnp.float32),
            pltpu.VMEM((_CHUNK,), jnp.float32),
            pltpu.SemaphoreType.DMA,
            pltpu.SemaphoreType.DMA,
        ],
    )
    return fn(u_pos_t, i_pos_t, u_neg_t, i_neg_t, *idx_arrays)


def _loss_body(pos_ref, neg_ref, out_ref):
    pos = pos_ref[...]
    neg = neg_ref[...]
    total = jnp.sum(jax.nn.log_sigmoid(pos) + jax.nn.log_sigmoid(-neg))
    out_ref[0, 0] = -total / _N_PAIRS


def _loss(pos_scores, neg_scores):
    p = pos_scores.reshape(_N_PAIRS // 128, 128)
    n = neg_scores.reshape(_N_PAIRS // 128, 128)
    out = pl.pallas_call(
        _loss_body,
        out_shape=jax.ShapeDtypeStruct((1, 1), jnp.float32),
        out_specs=pl.BlockSpec(memory_space=pltpu.SMEM),
    )(p, n)
    return out[0, 0]


def kernel(user_emb_pos, item_emb_pos, user_emb_neg, item_emb_neg,
           positive_pairs, negative_pairs):
    rs = lambda a: a.astype(jnp.int32).reshape(_NW, _NCHUNK, _CHUNK)
    pos_s, neg_s = _sc_scores(
        user_emb_pos, item_emb_pos, user_emb_neg, item_emb_neg,
        (rs(positive_pairs[:, 0]), rs(positive_pairs[:, 1]),
         rs(negative_pairs[:, 0]), rs(negative_pairs[:, 1])))
    return _loss(pos_s, neg_s)
